# baseline (speedup 1.0000x reference)
import jax
import jax.numpy as jnp
from jax.experimental import pallas as pl
from jax.experimental.pallas import tpu as pltpu
from functools import partial


def _proj_kernel(x_ref, aew_ref, aeb_ref, wl_ref, bl_ref, wr_ref, br_ref,
                 h_ref, xl_ref, xr_ref):
    x = x_ref[...]
    h = jax.lax.dot_general(x, aew_ref[...], (((1,), (0,)), ((), ())),
                            preferred_element_type=jnp.float32) + aeb_ref[...]
    h_ref[...] = h
    xl_ref[...] = jax.lax.dot_general(h, wl_ref[...], (((1,), (0,)), ((), ())),
                                      preferred_element_type=jnp.float32) + bl_ref[...]
    xr_ref[...] = jax.lax.dot_general(h, wr_ref[...], (((1,), (0,)), ((), ())),
                                      preferred_element_type=jnp.float32) + br_ref[...]


def _edge_kernel(src_ref, dst_ref, xl_ref, xr_ref, pa_ref, pt_ref, pd_ref,
                 msg_ref, den_ref, *, chunk):
    @pl.when(pl.program_id(0) == 0)
    def _init():
        msg_ref[...] = jnp.zeros(msg_ref.shape, jnp.float32)
        den_ref[...] = jnp.zeros(den_ref.shape, jnp.float32)

    def body(i, carry):
        s = src_ref[0, 0, i]
        d = dst_ref[0, 0, i]
        xlr = xl_ref[pl.ds(s, 1), :]
        xrr = xr_ref[pl.ds(d, 1), :]
        m = xlr + xrr
        m = jnp.where(m >= 0, m, 0.2 * m)
        alpha = jax.lax.dot_general(m, pa_ref[...], (((1,), (0,)), ((), ())),
                                    preferred_element_type=jnp.float32)
        w = jnp.exp(alpha)
        wb = jax.lax.dot_general(w, pt_ref[...], (((1,), (0,)), ((), ())),
                                 preferred_element_type=jnp.float32)
        msg_ref[pl.ds(d, 1), :] += xlr * wb
        den_ref[pl.ds(d, 1), :] += jax.lax.dot_general(
            w, pd_ref[...], (((1,), (0,)), ((), ())),
            preferred_element_type=jnp.float32)
        return carry

    jax.lax.fori_loop(0, chunk, body, 0)


def _final_kernel(msg_ref, den_ref, h_ref, q_ref, gb_ref, lw_ref, lb_ref,
                  lnw_ref, lnb_ref, out_ref):
    den = jax.lax.dot_general(den_ref[...], q_ref[...], (((1,), (0,)), ((), ())),
                              preferred_element_type=jnp.float32)
    a = msg_ref[...] / (den + 1e-16) + gb_ref[...]
    y = jax.lax.dot_general(a, lw_ref[...], (((1,), (0,)), ((), ())),
                            preferred_element_type=jnp.float32) + lb_ref[...]
    y = jnp.maximum(y, 0.0)
    r = y + h_ref[...]
    mu = jnp.mean(r, axis=1, keepdims=True)
    var = jnp.mean((r - mu) ** 2, axis=1, keepdims=True)
    out_ref[...] = lnw_ref[...] * ((r - mu) * jax.lax.rsqrt(var + 1e-12)) + lnb_ref[...]


def kernel(x, edge_index, AE_W, AE_b, Wl, bl, Wr, br, att, gat_bias,
           lin_W, lin_b, ln_w, ln_b):
    n, d = x.shape
    hh, c = att.shape
    hc = hh * c
    e = edge_index.shape[1]
    ep = e + n

    nb = 1000 if n % 1000 == 0 else n
    h, xl, xr = pl.pallas_call(
        _proj_kernel,
        grid=(n // nb,),
        in_specs=[
            pl.BlockSpec((nb, d), lambda i: (i, 0)),
            pl.BlockSpec((d, d), lambda i: (0, 0)),
            pl.BlockSpec((1, d), lambda i: (0, 0)),
            pl.BlockSpec((d, hc), lambda i: (0, 0)),
            pl.BlockSpec((1, hc), lambda i: (0, 0)),
            pl.BlockSpec((d, hc), lambda i: (0, 0)),
            pl.BlockSpec((1, hc), lambda i: (0, 0)),
        ],
        out_specs=[
            pl.BlockSpec((nb, d), lambda i: (i, 0)),
            pl.BlockSpec((nb, hc), lambda i: (i, 0)),
            pl.BlockSpec((nb, hc), lambda i: (i, 0)),
        ],
        out_shape=[
            jax.ShapeDtypeStruct((n, d), jnp.float32),
            jax.ShapeDtypeStruct((n, hc), jnp.float32),
            jax.ShapeDtypeStruct((n, hc), jnp.float32),
        ],
    )(x, AE_W, AE_b.reshape(1, d), Wl, bl.reshape(1, hc), Wr, br.reshape(1, hc))

    loop = jnp.arange(n, dtype=edge_index.dtype)
    src = jnp.concatenate([edge_index[0], loop])
    dst = jnp.concatenate([edge_index[1], loop])
    chunk = 1000 if ep % 1000 == 0 else ep
    nchunk = ep // chunk
    src3 = src.reshape(nchunk, 1, chunk)
    dst3 = dst.reshape(nchunk, 1, chunk)

    headmask = jnp.kron(jnp.eye(hh, dtype=jnp.float32),
                        jnp.ones((c, 1), jnp.float32))
    pa = headmask * att.reshape(hc, 1)
    pt = headmask.T
    pd = jnp.eye(hh, d, dtype=jnp.float32)

    msg, den = pl.pallas_call(
        partial(_edge_kernel, chunk=chunk),
        grid=(nchunk,),
        compiler_params=pltpu.CompilerParams(vmem_limit_bytes=100 * 1024 * 1024),
        in_specs=[
            pl.BlockSpec((1, 1, chunk), lambda i: (i, 0, 0),
                         memory_space=pltpu.SMEM),
            pl.BlockSpec((1, 1, chunk), lambda i: (i, 0, 0),
                         memory_space=pltpu.SMEM),
            pl.BlockSpec((n, hc), lambda i: (0, 0)),
            pl.BlockSpec((n, hc), lambda i: (0, 0)),
            pl.BlockSpec((hc, hh), lambda i: (0, 0)),
            pl.BlockSpec((hh, hc), lambda i: (0, 0)),
            pl.BlockSpec((hh, d), lambda i: (0, 0)),
        ],
        out_specs=[
            pl.BlockSpec((n, hc), lambda i: (0, 0)),
            pl.BlockSpec((n, d), lambda i: (0, 0)),
        ],
        out_shape=[
            jax.ShapeDtypeStruct((n, hc), jnp.float32),
            jax.ShapeDtypeStruct((n, d), jnp.float32),
        ],
    )(src3, dst3, xl, xr, pa, pt, pd)

    q = jnp.kron(jnp.eye(hh, dtype=jnp.float32),
                 jnp.ones((1, c), jnp.float32))
    q = jnp.concatenate([q, jnp.zeros((d - hh, hc), jnp.float32)], axis=0)

    out = pl.pallas_call(
        _final_kernel,
        grid=(n // nb,),
        in_specs=[
            pl.BlockSpec((nb, hc), lambda i: (i, 0)),
            pl.BlockSpec((nb, d), lambda i: (i, 0)),
            pl.BlockSpec((nb, d), lambda i: (i, 0)),
            pl.BlockSpec((d, hc), lambda i: (0, 0)),
            pl.BlockSpec((1, hc), lambda i: (0, 0)),
            pl.BlockSpec((hc, d), lambda i: (0, 0)),
            pl.BlockSpec((1, d), lambda i: (0, 0)),
            pl.BlockSpec((1, d), lambda i: (0, 0)),
            pl.BlockSpec((1, d), lambda i: (0, 0)),
        ],
        out_specs=pl.BlockSpec((nb, d), lambda i: (i, 0)),
        out_shape=jax.ShapeDtypeStruct((n, d), jnp.float32),
    )(msg, den, h, q, gat_bias.reshape(1, hc), lin_W, lin_b.reshape(1, d),
      ln_w.reshape(1, d), ln_b.reshape(1, d))
    return out


# speedup vs baseline: 1.6737x; 1.6737x over previous
import jax
import jax.numpy as jnp
from jax.experimental import pallas as pl
from jax.experimental.pallas import tpu as pltpu
from functools import partial


def _proj_kernel(x_ref, aew_ref, aeb_ref, wl_ref, bl_ref, wr_ref, br_ref,
                 h_ref, xl_ref, xr_ref):
    x = x_ref[...]
    h = jax.lax.dot_general(x, aew_ref[...], (((1,), (0,)), ((), ())),
                            preferred_element_type=jnp.float32) + aeb_ref[...]
    h_ref[...] = h
    xl_ref[...] = jax.lax.dot_general(h, wl_ref[...], (((1,), (0,)), ((), ())),
                                      preferred_element_type=jnp.float32) + bl_ref[...]
    xr_ref[...] = jax.lax.dot_general(h, wr_ref[...], (((1,), (0,)), ((), ())),
                                      preferred_element_type=jnp.float32) + br_ref[...]


def _edge_kernel(src_ref, dst_ref, xl_ref, xr_ref, pa_ref, pt_ref, pd_ref,
                 msg_ref, den_ref, *, chunk):
    @pl.when(pl.program_id(0) == 0)
    def _init():
        msg_ref[...] = jnp.zeros(msg_ref.shape, jnp.float32)
        den_ref[...] = jnp.zeros(den_ref.shape, jnp.float32)

    def body(i, carry):
        s = src_ref[0, 0, i]
        d = dst_ref[0, 0, i]
        xlr = xl_ref[pl.ds(s, 1), :]
        xrr = xr_ref[pl.ds(d, 1), :]
        m = xlr + xrr
        m = jnp.where(m >= 0, m, 0.2 * m)
        alpha = jax.lax.dot_general(m, pa_ref[...], (((1,), (0,)), ((), ())),
                                    preferred_element_type=jnp.float32)
        w = jnp.exp(alpha)
        wb = jax.lax.dot_general(w, pt_ref[...], (((1,), (0,)), ((), ())),
                                 preferred_element_type=jnp.float32)
        msg_ref[pl.ds(d, 1), :] += xlr * wb
        den_ref[pl.ds(d, 1), :] += jax.lax.dot_general(
            w, pd_ref[...], (((1,), (0,)), ((), ())),
            preferred_element_type=jnp.float32)
        return carry

    jax.lax.fori_loop(0, chunk, body, 0, unroll=8)


def _final_kernel(msg_ref, den_ref, h_ref, q_ref, gb_ref, lw_ref, lb_ref,
                  lnw_ref, lnb_ref, out_ref):
    den = jax.lax.dot_general(den_ref[...], q_ref[...], (((1,), (0,)), ((), ())),
                              preferred_element_type=jnp.float32)
    a = msg_ref[...] / (den + 1e-16) + gb_ref[...]
    y = jax.lax.dot_general(a, lw_ref[...], (((1,), (0,)), ((), ())),
                            preferred_element_type=jnp.float32) + lb_ref[...]
    y = jnp.maximum(y, 0.0)
    r = y + h_ref[...]
    mu = jnp.mean(r, axis=1, keepdims=True)
    var = jnp.mean((r - mu) ** 2, axis=1, keepdims=True)
    out_ref[...] = lnw_ref[...] * ((r - mu) * jax.lax.rsqrt(var + 1e-12)) + lnb_ref[...]


def kernel(x, edge_index, AE_W, AE_b, Wl, bl, Wr, br, att, gat_bias,
           lin_W, lin_b, ln_w, ln_b):
    n, d = x.shape
    hh, c = att.shape
    hc = hh * c
    e = edge_index.shape[1]
    ep = e + n

    nb = 1000 if n % 1000 == 0 else n
    h, xl, xr = pl.pallas_call(
        _proj_kernel,
        grid=(n // nb,),
        in_specs=[
            pl.BlockSpec((nb, d), lambda i: (i, 0)),
            pl.BlockSpec((d, d), lambda i: (0, 0)),
            pl.BlockSpec((1, d), lambda i: (0, 0)),
            pl.BlockSpec((d, hc), lambda i: (0, 0)),
            pl.BlockSpec((1, hc), lambda i: (0, 0)),
            pl.BlockSpec((d, hc), lambda i: (0, 0)),
            pl.BlockSpec((1, hc), lambda i: (0, 0)),
        ],
        out_specs=[
            pl.BlockSpec((nb, d), lambda i: (i, 0)),
            pl.BlockSpec((nb, hc), lambda i: (i, 0)),
            pl.BlockSpec((nb, hc), lambda i: (i, 0)),
        ],
        out_shape=[
            jax.ShapeDtypeStruct((n, d), jnp.float32),
            jax.ShapeDtypeStruct((n, hc), jnp.float32),
            jax.ShapeDtypeStruct((n, hc), jnp.float32),
        ],
    )(x, AE_W, AE_b.reshape(1, d), Wl, bl.reshape(1, hc), Wr, br.reshape(1, hc))

    loop = jnp.arange(n, dtype=edge_index.dtype)
    src = jnp.concatenate([edge_index[0], loop])
    dst = jnp.concatenate([edge_index[1], loop])
    chunk = 1000 if ep % 1000 == 0 else ep
    nchunk = ep // chunk
    src3 = src.reshape(nchunk, 1, chunk)
    dst3 = dst.reshape(nchunk, 1, chunk)

    headmask = jnp.kron(jnp.eye(hh, dtype=jnp.float32),
                        jnp.ones((c, 1), jnp.float32))
    pa = headmask * att.reshape(hc, 1)
    pt = headmask.T
    pd = jnp.eye(hh, d, dtype=jnp.float32)

    msg, den = pl.pallas_call(
        partial(_edge_kernel, chunk=chunk),
        grid=(nchunk,),
        compiler_params=pltpu.CompilerParams(vmem_limit_bytes=100 * 1024 * 1024),
        in_specs=[
            pl.BlockSpec((1, 1, chunk), lambda i: (i, 0, 0),
                         memory_space=pltpu.SMEM),
            pl.BlockSpec((1, 1, chunk), lambda i: (i, 0, 0),
                         memory_space=pltpu.SMEM),
            pl.BlockSpec((n, hc), lambda i: (0, 0)),
            pl.BlockSpec((n, hc), lambda i: (0, 0)),
            pl.BlockSpec((hc, hh), lambda i: (0, 0)),
            pl.BlockSpec((hh, hc), lambda i: (0, 0)),
            pl.BlockSpec((hh, d), lambda i: (0, 0)),
        ],
        out_specs=[
            pl.BlockSpec((n, hc), lambda i: (0, 0)),
            pl.BlockSpec((n, d), lambda i: (0, 0)),
        ],
        out_shape=[
            jax.ShapeDtypeStruct((n, hc), jnp.float32),
            jax.ShapeDtypeStruct((n, d), jnp.float32),
        ],
    )(src3, dst3, xl, xr, pa, pt, pd)

    q = jnp.kron(jnp.eye(hh, dtype=jnp.float32),
                 jnp.ones((1, c), jnp.float32))
    q = jnp.concatenate([q, jnp.zeros((d - hh, hc), jnp.float32)], axis=0)

    out = pl.pallas_call(
        _final_kernel,
        grid=(n // nb,),
        in_specs=[
            pl.BlockSpec((nb, hc), lambda i: (i, 0)),
            pl.BlockSpec((nb, d), lambda i: (i, 0)),
            pl.BlockSpec((nb, d), lambda i: (i, 0)),
            pl.BlockSpec((d, hc), lambda i: (0, 0)),
            pl.BlockSpec((1, hc), lambda i: (0, 0)),
            pl.BlockSpec((hc, d), lambda i: (0, 0)),
            pl.BlockSpec((1, d), lambda i: (0, 0)),
            pl.BlockSpec((1, d), lambda i: (0, 0)),
            pl.BlockSpec((1, d), lambda i: (0, 0)),
        ],
        out_specs=pl.BlockSpec((nb, d), lambda i: (i, 0)),
        out_shape=jax.ShapeDtypeStruct((n, d), jnp.float32),
    )(msg, den, h, q, gat_bias.reshape(1, hc), lin_W, lin_b.reshape(1, d),
      ln_w.reshape(1, d), ln_b.reshape(1, d))
    return out


# speedup vs baseline: 1.7573x; 1.0500x over previous
import jax
import jax.numpy as jnp
from jax.experimental import pallas as pl
from jax.experimental.pallas import tpu as pltpu
from functools import partial


def _proj_kernel(x_ref, aew_ref, aeb_ref, wl_ref, bl_ref, wr_ref, br_ref,
                 h_ref, xl_ref, xr_ref):
    x = x_ref[...]
    h = jax.lax.dot_general(x, aew_ref[...], (((1,), (0,)), ((), ())),
                            preferred_element_type=jnp.float32) + aeb_ref[...]
    h_ref[...] = h
    xl_ref[...] = jax.lax.dot_general(h, wl_ref[...], (((1,), (0,)), ((), ())),
                                      preferred_element_type=jnp.float32) + bl_ref[...]
    xr_ref[...] = jax.lax.dot_general(h, wr_ref[...], (((1,), (0,)), ((), ())),
                                      preferred_element_type=jnp.float32) + br_ref[...]


def _edge_kernel(src_ref, dst_ref, xl_ref, xr_ref, pa_ref, pt_ref, pd_ref,
                 msg_ref, den_ref, *, chunk):
    @pl.when(pl.program_id(0) == 0)
    def _init():
        msg_ref[...] = jnp.zeros(msg_ref.shape, jnp.float32)
        den_ref[...] = jnp.zeros(den_ref.shape, jnp.float32)

    def body(i, carry):
        s = src_ref[0, 0, i]
        d = dst_ref[0, 0, i]
        xlr = xl_ref[pl.ds(s, 1), :]
        xrr = xr_ref[pl.ds(d, 1), :]
        m = xlr + xrr
        m = jnp.where(m >= 0, m, 0.2 * m)
        alpha = jax.lax.dot_general(m, pa_ref[...], (((1,), (0,)), ((), ())),
                                    preferred_element_type=jnp.float32)
        w = jnp.exp(alpha)
        wb = jax.lax.dot_general(w, pt_ref[...], (((1,), (0,)), ((), ())),
                                 preferred_element_type=jnp.float32)
        msg_ref[pl.ds(d, 1), :] += xlr * wb
        den_ref[pl.ds(d, 1), :] += jax.lax.dot_general(
            w, pd_ref[...], (((1,), (0,)), ((), ())),
            preferred_element_type=jnp.float32)
        return carry

    jax.lax.fori_loop(0, chunk, body, 0, unroll=16)


def _final_kernel(msg_ref, den_ref, h_ref, q_ref, gb_ref, lw_ref, lb_ref,
                  lnw_ref, lnb_ref, out_ref):
    den = jax.lax.dot_general(den_ref[...], q_ref[...], (((1,), (0,)), ((), ())),
                              preferred_element_type=jnp.float32)
    a = msg_ref[...] / (den + 1e-16) + gb_ref[...]
    y = jax.lax.dot_general(a, lw_ref[...], (((1,), (0,)), ((), ())),
                            preferred_element_type=jnp.float32) + lb_ref[...]
    y = jnp.maximum(y, 0.0)
    r = y + h_ref[...]
    mu = jnp.mean(r, axis=1, keepdims=True)
    var = jnp.mean((r - mu) ** 2, axis=1, keepdims=True)
    out_ref[...] = lnw_ref[...] * ((r - mu) * jax.lax.rsqrt(var + 1e-12)) + lnb_ref[...]


def kernel(x, edge_index, AE_W, AE_b, Wl, bl, Wr, br, att, gat_bias,
           lin_W, lin_b, ln_w, ln_b):
    n, d = x.shape
    hh, c = att.shape
    hc = hh * c
    e = edge_index.shape[1]
    ep = e + n

    nb = 1000 if n % 1000 == 0 else n
    h, xl, xr = pl.pallas_call(
        _proj_kernel,
        grid=(n // nb,),
        in_specs=[
            pl.BlockSpec((nb, d), lambda i: (i, 0)),
            pl.BlockSpec((d, d), lambda i: (0, 0)),
            pl.BlockSpec((1, d), lambda i: (0, 0)),
            pl.BlockSpec((d, hc), lambda i: (0, 0)),
            pl.BlockSpec((1, hc), lambda i: (0, 0)),
            pl.BlockSpec((d, hc), lambda i: (0, 0)),
            pl.BlockSpec((1, hc), lambda i: (0, 0)),
        ],
        out_specs=[
            pl.BlockSpec((nb, d), lambda i: (i, 0)),
            pl.BlockSpec((nb, hc), lambda i: (i, 0)),
            pl.BlockSpec((nb, hc), lambda i: (i, 0)),
        ],
        out_shape=[
            jax.ShapeDtypeStruct((n, d), jnp.float32),
            jax.ShapeDtypeStruct((n, hc), jnp.float32),
            jax.ShapeDtypeStruct((n, hc), jnp.float32),
        ],
    )(x, AE_W, AE_b.reshape(1, d), Wl, bl.reshape(1, hc), Wr, br.reshape(1, hc))

    loop = jnp.arange(n, dtype=edge_index.dtype)
    src = jnp.concatenate([edge_index[0], loop])
    dst = jnp.concatenate([edge_index[1], loop])
    chunk = 1000 if ep % 1000 == 0 else ep
    nchunk = ep // chunk
    src3 = src.reshape(nchunk, 1, chunk)
    dst3 = dst.reshape(nchunk, 1, chunk)

    headmask = jnp.kron(jnp.eye(hh, dtype=jnp.float32),
                        jnp.ones((c, 1), jnp.float32))
    pa = headmask * att.reshape(hc, 1)
    pt = headmask.T
    pd = jnp.eye(hh, d, dtype=jnp.float32)

    msg, den = pl.pallas_call(
        partial(_edge_kernel, chunk=chunk),
        grid=(nchunk,),
        compiler_params=pltpu.CompilerParams(vmem_limit_bytes=100 * 1024 * 1024),
        in_specs=[
            pl.BlockSpec((1, 1, chunk), lambda i: (i, 0, 0),
                         memory_space=pltpu.SMEM),
            pl.BlockSpec((1, 1, chunk), lambda i: (i, 0, 0),
                         memory_space=pltpu.SMEM),
            pl.BlockSpec((n, hc), lambda i: (0, 0)),
            pl.BlockSpec((n, hc), lambda i: (0, 0)),
            pl.BlockSpec((hc, hh), lambda i: (0, 0)),
            pl.BlockSpec((hh, hc), lambda i: (0, 0)),
            pl.BlockSpec((hh, d), lambda i: (0, 0)),
        ],
        out_specs=[
            pl.BlockSpec((n, hc), lambda i: (0, 0)),
            pl.BlockSpec((n, d), lambda i: (0, 0)),
        ],
        out_shape=[
            jax.ShapeDtypeStruct((n, hc), jnp.float32),
            jax.ShapeDtypeStruct((n, d), jnp.float32),
        ],
    )(src3, dst3, xl, xr, pa, pt, pd)

    q = jnp.kron(jnp.eye(hh, dtype=jnp.float32),
                 jnp.ones((1, c), jnp.float32))
    q = jnp.concatenate([q, jnp.zeros((d - hh, hc), jnp.float32)], axis=0)

    out = pl.pallas_call(
        _final_kernel,
        grid=(n // nb,),
        in_specs=[
            pl.BlockSpec((nb, hc), lambda i: (i, 0)),
            pl.BlockSpec((nb, d), lambda i: (i, 0)),
            pl.BlockSpec((nb, d), lambda i: (i, 0)),
            pl.BlockSpec((d, hc), lambda i: (0, 0)),
            pl.BlockSpec((1, hc), lambda i: (0, 0)),
            pl.BlockSpec((hc, d), lambda i: (0, 0)),
            pl.BlockSpec((1, d), lambda i: (0, 0)),
            pl.BlockSpec((1, d), lambda i: (0, 0)),
            pl.BlockSpec((1, d), lambda i: (0, 0)),
        ],
        out_specs=pl.BlockSpec((nb, d), lambda i: (i, 0)),
        out_shape=jax.ShapeDtypeStruct((n, d), jnp.float32),
    )(msg, den, h, q, gat_bias.reshape(1, hc), lin_W, lin_b.reshape(1, d),
      ln_w.reshape(1, d), ln_b.reshape(1, d))
    return out
